# baseline (speedup 1.0000x reference)
import jax
import jax.numpy as jnp
from jax import lax
from jax.experimental import pallas as pl

EPS = 1e-5


def _conv(x, w, b, stride=(1, 1), padding=((0, 0), (0, 0))):
    out = lax.conv_general_dilated(x, w, window_strides=stride, padding=padding,
                                   dimension_numbers=('NCHW', 'OIHW', 'NCHW'))
    return out + b[None, :, None, None]


def _conv_t(x, w, b, stride, kernel, padding, out_pad):
    kh, kw = kernel
    ph, pw = padding
    oph, opw = out_pad
    pads = ((kh - 1 - ph, kh - 1 - ph + oph), (kw - 1 - pw, kw - 1 - pw + opw))
    out = lax.conv_general_dilated(x, w, window_strides=(1, 1), padding=pads,
                                   lhs_dilation=stride,
                                   dimension_numbers=('NCHW', 'OIHW', 'NCHW'))
    return out + b[None, :, None, None]


def _conv_t_fast(x, w, b, kernel, padding, out_pad):
    kh, kw = kernel
    ph, pw = padding
    oph, opw = out_pad
    B, Cin, Hin, Win = x.shape
    pl_h, pr_h = kh - 1 - ph, kh - 1 - ph + oph
    pl_w, pr_w = kw - 1 - pw, kw - 1 - pw + opw
    H_out = (2 * Hin - 1) + pl_h + pr_h - kh + 1
    W_out = (2 * Win - 1) + pl_w + pr_w - kw + 1

    def phase_1d(ph0, k_len, pl, n_in, n_out):
        ks = [k for k in range(k_len) if (ph0 + k - pl) % 2 == 0]
        L = (n_out - ph0 + 1) // 2
        off0 = (ph0 + ks[0] - pl) // 2
        pad_l = -off0
        pad_r = L - 1 + len(ks) - n_in - pad_l
        return ks, L, pad_l, pad_r

    cols = {}
    for r in (0, 1):
        ks_h, L_h, plh, prh = phase_1d(r, kh, pl_h, Hin, H_out)
        for s in (0, 1):
            ks_w, L_w, plw, prw = phase_1d(s, kw, pl_w, Win, W_out)
            sub_w = w[:, :, ks_h, :][:, :, :, ks_w]
            cols[(r, s)] = (lax.conv_general_dilated(
                x, sub_w, window_strides=(1, 1),
                padding=((plh, prh), (plw, prw)),
                dimension_numbers=('NCHW', 'OIHW', 'NCHW')), L_w)

    rows = []
    for r in (0, 1):
        (a, L_w0), (bb, L_w1) = cols[(r, 0)], cols[(r, 1)]
        L_w = max(L_w0, L_w1)
        if a.shape[3] < L_w:
            a = jnp.pad(a, ((0, 0), (0, 0), (0, 0), (0, L_w - a.shape[3])))
        if bb.shape[3] < L_w:
            bb = jnp.pad(bb, ((0, 0), (0, 0), (0, 0), (0, L_w - bb.shape[3])))
        row = jnp.stack([a, bb], axis=-1).reshape(a.shape[0], a.shape[1],
                                                  a.shape[2], 2 * L_w)
        rows.append(row[:, :, :, :W_out])
    a, bb = rows
    L_h = max(a.shape[2], bb.shape[2])
    if a.shape[2] < L_h:
        a = jnp.pad(a, ((0, 0), (0, 0), (0, L_h - a.shape[2]), (0, 0)))
    if bb.shape[2] < L_h:
        bb = jnp.pad(bb, ((0, 0), (0, 0), (0, L_h - bb.shape[2]), (0, 0)))
    out = jnp.stack([a, bb], axis=3).reshape(a.shape[0], a.shape[1],
                                             2 * L_h, a.shape[3])
    out = out[:, :, :H_out, :]
    return out + b[None, :, None, None]


def _bn(x, g, b):
    m = x.mean(axis=(0, 2, 3), keepdims=True)
    v = x.var(axis=(0, 2, 3), keepdims=True)
    return g[None, :, None, None] * (x - m) * lax.rsqrt(v + EPS) + b[None, :, None, None]


def _res(x, w1, b1, w2, b2):
    h = jax.nn.relu(x)
    h = _conv(h, w1, b1, (1, 1), ((1, 1), (1, 1)))
    h = jax.nn.relu(h)
    h = _conv(h, w2, b2)
    return x + h


def _vq_body(h_ref, e_ref, et_ref, w2_ref, b2_ref, wp_ref, bp_ref,
             out_ref, loss_ref):
    h = h_ref[0]
    z = jnp.dot(w2_ref[:], h, preferred_element_type=jnp.float32) + b2_ref[:]
    et = et_ref[:]
    e2 = jnp.sum(et * et, axis=1, keepdims=True)
    scores = e2 - 2.0 * jnp.dot(et, z, preferred_element_type=jnp.float32)
    idx = jnp.argmin(scores, axis=0)
    onehot = (lax.broadcasted_iota(jnp.int32, scores.shape, 0)
              == idx[None, :]).astype(jnp.float32)
    quant = jnp.dot(e_ref[:], onehot, preferred_element_type=jnp.float32)
    d = quant - z
    part = jnp.sum(d * d).reshape(1, 1)
    out_ref[0] = jnp.dot(wp_ref[:], quant,
                         preferred_element_type=jnp.float32) + bp_ref[:]

    @pl.when(pl.program_id(0) == 0)
    def _():
        loss_ref[...] = jnp.zeros((1, 1), jnp.float32)

    loss_ref[...] += part


def _vq_pallas(h, E, w2, b2, wp, bp):
    B, D, S = h.shape
    K = E.shape[1]
    grid = (B,)
    out, loss_sum = pl.pallas_call(
        _vq_body,
        grid=grid,
        in_specs=[
            pl.BlockSpec((1, D, S), lambda i: (i, 0, 0)),
            pl.BlockSpec((D, K), lambda i: (0, 0)),
            pl.BlockSpec((K, D), lambda i: (0, 0)),
            pl.BlockSpec((D, D), lambda i: (0, 0)),
            pl.BlockSpec((D, 1), lambda i: (0, 0)),
            pl.BlockSpec((D, D), lambda i: (0, 0)),
            pl.BlockSpec((D, 1), lambda i: (0, 0)),
        ],
        out_specs=[
            pl.BlockSpec((1, D, S), lambda i: (i, 0, 0)),
            pl.BlockSpec((1, 1), lambda i: (0, 0)),
        ],
        out_shape=[
            jax.ShapeDtypeStruct((B, D, S), jnp.float32),
            jax.ShapeDtypeStruct((1, 1), jnp.float32),
        ],
    )(h, E, E.T, w2, b2, wp, bp)
    return out, loss_sum[0, 0]


def kernel(x, params):
    p = params
    h = _conv(x, p['enc_w1'], p['enc_b1'], (2, 2), ((1, 1), (1, 1)))
    h = jax.nn.relu(_bn(h, p['enc_g1'], p['enc_be1']))
    h = _conv(h, p['enc_w2'], p['enc_b2'], (2, 2), ((1, 1), (1, 1)))
    h = jax.nn.relu(_bn(h, p['enc_g2'], p['enc_be2']))
    h = _conv(h, p['enc_w3'], p['enc_b3'])
    h = _conv(h, p['pre_w1'], p['pre_b1'])
    h = _res(h, p['pre_r1_w1'], p['pre_r1_b1'], p['pre_r1_w2'], p['pre_r1_b2'])
    h = _res(h, p['pre_r2_w1'], p['pre_r2_b1'], p['pre_r2_w2'], p['pre_r2_b2'])

    E = p['embedding']
    B, D, H, W = h.shape
    out, loss_sum = _vq_pallas(
        h.reshape(B, D, H * W), E,
        p['pre_w2'][:, :, 0, 0], p['pre_b2'][:, None],
        p['post_w1'][:, :, 0, 0], p['post_b1'][:, None])
    loss = 1.25 * loss_sum / (B * D * H * W)
    h = out.reshape(B, D, H, W)

    h = _res(h, p['post_r1_w1'], p['post_r1_b1'], p['post_r1_w2'], p['post_r1_b2'])
    h = _res(h, p['post_r2_w1'], p['post_r2_b1'], p['post_r2_w2'], p['post_r2_b2'])
    h = _conv(h, p['post_w2'], p['post_b2'])
    h = _conv_t_fast(h, p['dec_w1'], p['dec_b1'], (4, 3), (1, 1), (0, 0))
    h = jax.nn.relu(_bn(h, p['dec_g1'], p['dec_be1']))
    recon = _conv_t_fast(h, p['dec_w2'], p['dec_b2'], (4, 3), (1, 1), (0, 1))
    return recon, loss


# speedup vs baseline: 3.0382x; 3.0382x over previous
import jax
import jax.numpy as jnp
from jax import lax
from jax.experimental import pallas as pl

EPS = 1e-5


def _conv(x, w, b, stride=(1, 1), padding=((0, 0), (0, 0))):
    out = lax.conv_general_dilated(x, w, window_strides=stride, padding=padding,
                                   dimension_numbers=('NCHW', 'OIHW', 'NCHW'))
    return out + b[None, :, None, None]


def _conv_t(x, w, b, stride, kernel, padding, out_pad):
    kh, kw = kernel
    ph, pw = padding
    oph, opw = out_pad
    pads = ((kh - 1 - ph, kh - 1 - ph + oph), (kw - 1 - pw, kw - 1 - pw + opw))
    out = lax.conv_general_dilated(x, w, window_strides=(1, 1), padding=pads,
                                   lhs_dilation=stride,
                                   dimension_numbers=('NCHW', 'OIHW', 'NCHW'))
    return out + b[None, :, None, None]


def _conv_b(x, w, b, stride=(1, 1), padding=((0, 0), (0, 0))):
    out = lax.conv_general_dilated(x.astype(jnp.bfloat16), w.astype(jnp.bfloat16),
                                   window_strides=stride, padding=padding,
                                   dimension_numbers=('NCHW', 'OIHW', 'NCHW'),
                                   preferred_element_type=jnp.float32)
    return out + b[None, :, None, None]


def _conv_t_b(x, w, b, stride, kernel, padding, out_pad):
    kh, kw = kernel
    ph, pw = padding
    oph, opw = out_pad
    pads = ((kh - 1 - ph, kh - 1 - ph + oph), (kw - 1 - pw, kw - 1 - pw + opw))
    out = lax.conv_general_dilated(x.astype(jnp.bfloat16), w.astype(jnp.bfloat16),
                                   window_strides=(1, 1), padding=pads,
                                   lhs_dilation=stride,
                                   dimension_numbers=('NCHW', 'OIHW', 'NCHW'),
                                   preferred_element_type=jnp.float32)
    return out + b[None, :, None, None]


def _res_b(x, w1, b1, w2, b2):
    h = jax.nn.relu(x)
    h = _conv_b(h, w1, b1, (1, 1), ((1, 1), (1, 1)))
    h = jax.nn.relu(h)
    h = _conv_b(h, w2, b2)
    return x + h


def _bn(x, g, b):
    m = x.mean(axis=(0, 2, 3), keepdims=True)
    v = x.var(axis=(0, 2, 3), keepdims=True)
    return g[None, :, None, None] * (x - m) * lax.rsqrt(v + EPS) + b[None, :, None, None]


def _res(x, w1, b1, w2, b2):
    h = jax.nn.relu(x)
    h = _conv(h, w1, b1, (1, 1), ((1, 1), (1, 1)))
    h = jax.nn.relu(h)
    h = _conv(h, w2, b2)
    return x + h


def _vq_body(h_ref, e_ref, et_ref, w2_ref, b2_ref, wp_ref, bp_ref,
             out_ref, loss_ref):
    h = h_ref[0]
    z = jnp.dot(w2_ref[:], h, preferred_element_type=jnp.float32) + b2_ref[:]
    et = et_ref[:]
    e2 = jnp.sum(et * et, axis=1, keepdims=True)
    scores = e2 - 2.0 * jnp.dot(et, z, preferred_element_type=jnp.float32)
    idx = jnp.argmin(scores, axis=0)
    onehot = (lax.broadcasted_iota(jnp.int32, scores.shape, 0)
              == idx[None, :]).astype(jnp.float32)
    quant = jnp.dot(e_ref[:], onehot, preferred_element_type=jnp.float32)
    d = quant - z
    part = jnp.sum(d * d).reshape(1, 1)
    out_ref[0] = jnp.dot(wp_ref[:], quant,
                         preferred_element_type=jnp.float32) + bp_ref[:]

    @pl.when(pl.program_id(0) == 0)
    def _():
        loss_ref[...] = jnp.zeros((1, 1), jnp.float32)

    loss_ref[...] += part


def _vq_pallas(h, E, w2, b2, wp, bp):
    B, D, S = h.shape
    K = E.shape[1]
    grid = (B,)
    out, loss_sum = pl.pallas_call(
        _vq_body,
        grid=grid,
        in_specs=[
            pl.BlockSpec((1, D, S), lambda i: (i, 0, 0)),
            pl.BlockSpec((D, K), lambda i: (0, 0)),
            pl.BlockSpec((K, D), lambda i: (0, 0)),
            pl.BlockSpec((D, D), lambda i: (0, 0)),
            pl.BlockSpec((D, 1), lambda i: (0, 0)),
            pl.BlockSpec((D, D), lambda i: (0, 0)),
            pl.BlockSpec((D, 1), lambda i: (0, 0)),
        ],
        out_specs=[
            pl.BlockSpec((1, D, S), lambda i: (i, 0, 0)),
            pl.BlockSpec((1, 1), lambda i: (0, 0)),
        ],
        out_shape=[
            jax.ShapeDtypeStruct((B, D, S), jnp.float32),
            jax.ShapeDtypeStruct((1, 1), jnp.float32),
        ],
    )(h, E, E.T, w2, b2, wp, bp)
    return out, loss_sum[0, 0]


def kernel(x, params):
    p = params
    h = _conv(x, p['enc_w1'], p['enc_b1'], (2, 2), ((1, 1), (1, 1)))
    h = jax.nn.relu(_bn(h, p['enc_g1'], p['enc_be1']))
    h = _conv(h, p['enc_w2'], p['enc_b2'], (2, 2), ((1, 1), (1, 1)))
    h = jax.nn.relu(_bn(h, p['enc_g2'], p['enc_be2']))
    h = _conv(h, p['enc_w3'], p['enc_b3'])
    h = _conv(h, p['pre_w1'], p['pre_b1'])
    h = _res(h, p['pre_r1_w1'], p['pre_r1_b1'], p['pre_r1_w2'], p['pre_r1_b2'])
    h = _res(h, p['pre_r2_w1'], p['pre_r2_b1'], p['pre_r2_w2'], p['pre_r2_b2'])

    E = p['embedding']
    B, D, H, W = h.shape
    out, loss_sum = _vq_pallas(
        h.reshape(B, D, H * W), E,
        p['pre_w2'][:, :, 0, 0], p['pre_b2'][:, None],
        p['post_w1'][:, :, 0, 0], p['post_b1'][:, None])
    loss = 1.25 * loss_sum / (B * D * H * W)
    h = out.reshape(B, D, H, W)

    h = _res_b(h, p['post_r1_w1'], p['post_r1_b1'], p['post_r1_w2'], p['post_r1_b2'])
    h = _res_b(h, p['post_r2_w1'], p['post_r2_b1'], p['post_r2_w2'], p['post_r2_b2'])
    h = _conv_b(h, p['post_w2'], p['post_b2'])
    h = _conv_t_b(h, p['dec_w1'], p['dec_b1'], (2, 2), (4, 3), (1, 1), (0, 0))
    h = jax.nn.relu(_bn(h, p['dec_g1'], p['dec_be1']))
    recon = _conv_t_b(h, p['dec_w2'], p['dec_b2'], (2, 2), (4, 3), (1, 1), (0, 1))
    return recon, loss
